# baseline (speedup 1.0000x reference)
import jax
import jax.numpy as jnp
from jax.experimental import pallas as pl
from jax.experimental.pallas import tpu as pltpu


def _fused_body(x_ref, w_ref, adj_ref, b_ref, o_ref, s_ref):
    i = pl.program_id(1)

    @pl.when(i == 0)
    def _make_support():
        s_ref[...] = jnp.dot(
            x_ref[...], w_ref[...], preferred_element_type=jnp.float32
        )

    o_ref[...] = (
        jnp.dot(adj_ref[...], s_ref[...], preferred_element_type=jnp.float32)
        + b_ref[...]
    )


def _pick_tile(n, target):
    t = min(n, target)
    while t > 8 and (n % t or t % 8):
        t -= 8
    return t


def kernel(x, adj, weight, bias):
    N, Fin = x.shape
    Fout = weight.shape[1]
    f32 = jnp.float32

    x = x.astype(f32)
    adj = adj.astype(f32)
    weight = weight.astype(f32)
    b2 = bias.astype(f32).reshape(1, Fout)

    TM = _pick_tile(N, 256)
    n_tiles = N // TM
    n_cores = 2 if n_tiles % 2 == 0 else 1
    inner = n_tiles // n_cores

    out = pl.pallas_call(
        _fused_body,
        out_shape=jax.ShapeDtypeStruct((N, Fout), f32),
        grid=(n_cores, inner),
        in_specs=[
            pl.BlockSpec((N, Fin), lambda c, i: (0, 0)),
            pl.BlockSpec((Fin, Fout), lambda c, i: (0, 0)),
            pl.BlockSpec((TM, N), lambda c, i, _n=inner: (c * _n + i, 0)),
            pl.BlockSpec((1, Fout), lambda c, i: (0, 0)),
        ],
        out_specs=pl.BlockSpec((TM, Fout), lambda c, i, _n=inner: (c * _n + i, 0)),
        scratch_shapes=[pltpu.VMEM((N, Fout), f32)],
        compiler_params=pltpu.CompilerParams(
            dimension_semantics=("parallel", "arbitrary")),
        cost_estimate=pl.CostEstimate(
            flops=2 * N * N * Fout + 2 * n_cores * N * Fin * Fout,
            transcendentals=0,
            bytes_accessed=4 * (N * N + n_cores * N * Fin + N * Fout + Fout)),
    )(x, weight, adj, b2)

    return out


# speedup vs baseline: 1.1183x; 1.1183x over previous
import jax
import jax.numpy as jnp
from jax.experimental import pallas as pl
from jax.experimental.pallas import tpu as pltpu


def _fused_body(x_ref, w_ref, adj_ref, b_ref, o_ref, s_ref):
    i = pl.program_id(1)

    @pl.when(i == 0)
    def _make_support():
        s_ref[...] = jnp.dot(
            x_ref[...], w_ref[...], preferred_element_type=jnp.float32
        )

    o_ref[...] = (
        jnp.dot(adj_ref[...], s_ref[...], preferred_element_type=jnp.float32)
        + b_ref[...]
    )


def _pick_tile(n, target):
    t = min(n, target)
    while t > 8 and (n % t or t % 8):
        t -= 8
    return t


def kernel(x, adj, weight, bias):
    N, Fin = x.shape
    Fout = weight.shape[1]
    f32 = jnp.float32

    x = x.astype(f32)
    adj = adj.astype(f32)
    weight = weight.astype(f32)
    b2 = bias.astype(f32).reshape(1, Fout)

    TM = _pick_tile(N, 1024)
    n_tiles = N // TM
    n_cores = 2 if n_tiles % 2 == 0 else 1
    inner = n_tiles // n_cores

    out = pl.pallas_call(
        _fused_body,
        out_shape=jax.ShapeDtypeStruct((N, Fout), f32),
        grid=(n_cores, inner),
        in_specs=[
            pl.BlockSpec((N, Fin), lambda c, i: (0, 0)),
            pl.BlockSpec((Fin, Fout), lambda c, i: (0, 0)),
            pl.BlockSpec((TM, N), lambda c, i, _n=inner: (c * _n + i, 0)),
            pl.BlockSpec((1, Fout), lambda c, i: (0, 0)),
        ],
        out_specs=pl.BlockSpec((TM, Fout), lambda c, i, _n=inner: (c * _n + i, 0)),
        scratch_shapes=[pltpu.VMEM((N, Fout), f32)],
        compiler_params=pltpu.CompilerParams(
            dimension_semantics=("parallel", "arbitrary")),
        cost_estimate=pl.CostEstimate(
            flops=2 * N * N * Fout + 2 * n_cores * N * Fin * Fout,
            transcendentals=0,
            bytes_accessed=4 * (N * N + n_cores * N * Fin + N * Fout + Fout)),
    )(x, weight, adj, b2)

    return out


# speedup vs baseline: 1.1201x; 1.0016x over previous
import jax
import jax.numpy as jnp
from jax.experimental import pallas as pl
from jax.experimental.pallas import tpu as pltpu


def _fused_body(x_ref, w_ref, adj0_ref, adj1_ref, b_ref, o_ref, s_ref):
    i = pl.program_id(1)
    half = adj0_ref.shape[0]

    @pl.when(i == 0)
    def _make_support():
        s_ref[...] = jnp.dot(
            x_ref[...], w_ref[...], preferred_element_type=jnp.float32
        )

    o_ref[:half, :] = (
        jnp.dot(adj0_ref[...], s_ref[...], preferred_element_type=jnp.float32)
        + b_ref[...]
    )
    o_ref[half:, :] = (
        jnp.dot(adj1_ref[...], s_ref[...], preferred_element_type=jnp.float32)
        + b_ref[...]
    )


def _pick_tile(n, target):
    t = min(n, target)
    while t > 8 and (n % t or t % 8):
        t -= 8
    return t


def kernel(x, adj, weight, bias):
    N, Fin = x.shape
    Fout = weight.shape[1]
    f32 = jnp.float32

    x = x.astype(f32)
    adj = adj.astype(f32)
    weight = weight.astype(f32)
    b2 = bias.astype(f32).reshape(1, Fout)

    TM = _pick_tile(N, 512)
    n_tiles = N // TM
    n_cores = 2 if n_tiles % 2 == 0 else 1
    inner = n_tiles // n_cores
    half = TM // 2

    out = pl.pallas_call(
        _fused_body,
        out_shape=jax.ShapeDtypeStruct((N, Fout), f32),
        grid=(n_cores, inner),
        in_specs=[
            pl.BlockSpec((N, Fin), lambda c, i: (0, 0)),
            pl.BlockSpec((Fin, Fout), lambda c, i: (0, 0)),
            pl.BlockSpec((half, N), lambda c, i, _n=inner: (2 * (c * _n + i), 0)),
            pl.BlockSpec((half, N), lambda c, i, _n=inner: (2 * (c * _n + i) + 1, 0)),
            pl.BlockSpec((1, Fout), lambda c, i: (0, 0)),
        ],
        out_specs=pl.BlockSpec((TM, Fout), lambda c, i, _n=inner: (c * _n + i, 0)),
        scratch_shapes=[pltpu.VMEM((N, Fout), f32)],
        compiler_params=pltpu.CompilerParams(
            dimension_semantics=("parallel", "arbitrary")),
        cost_estimate=pl.CostEstimate(
            flops=2 * N * N * Fout + 2 * n_cores * N * Fin * Fout,
            transcendentals=0,
            bytes_accessed=4 * (N * N + n_cores * N * Fin + N * Fout + Fout)),
    )(x, weight, adj, adj, b2)

    return out


# speedup vs baseline: 1.1453x; 1.0224x over previous
import jax
import jax.numpy as jnp
from jax.experimental import pallas as pl
from jax.experimental.pallas import tpu as pltpu


def _fused_body(x_ref, w_ref, adj_ref, b_ref, o_ref, s_ref):
    i = pl.program_id(1)

    @pl.when(i == 0)
    def _make_support():
        s_ref[...] = jnp.dot(
            x_ref[...], w_ref[...], preferred_element_type=jnp.float32
        )

    o_ref[...] = (
        jnp.dot(adj_ref[...], s_ref[...], preferred_element_type=jnp.float32)
        + b_ref[...]
    )


def _pick_tile(n, target):
    t = min(n, target)
    while t > 8 and (n % t or t % 8):
        t -= 8
    return t


def kernel(x, adj, weight, bias):
    N, Fin = x.shape
    Fout = weight.shape[1]
    f32 = jnp.float32

    x = x.astype(f32)
    adj = adj.astype(f32)
    weight = weight.astype(f32)
    b2 = bias.astype(f32).reshape(1, Fout)

    TM = _pick_tile(N, 512)
    n_tiles = N // TM
    n_cores = 2 if n_tiles % 2 == 0 else 1
    inner = n_tiles // n_cores

    out = pl.pallas_call(
        _fused_body,
        out_shape=jax.ShapeDtypeStruct((N, Fout), f32),
        grid=(n_cores, inner),
        in_specs=[
            pl.BlockSpec((N, Fin), lambda c, i: (0, 0)),
            pl.BlockSpec((Fin, Fout), lambda c, i: (0, 0)),
            pl.BlockSpec((TM, N), lambda c, i, _n=inner: (c * _n + i, 0)),
            pl.BlockSpec((1, Fout), lambda c, i: (0, 0)),
        ],
        out_specs=pl.BlockSpec((TM, Fout), lambda c, i, _n=inner: (c * _n + i, 0)),
        scratch_shapes=[pltpu.VMEM((N, Fout), f32)],
        compiler_params=pltpu.CompilerParams(
            dimension_semantics=("parallel", "arbitrary")),
        cost_estimate=pl.CostEstimate(
            flops=2 * N * N * Fout + 2 * n_cores * N * Fin * Fout,
            transcendentals=0,
            bytes_accessed=4 * (N * N + n_cores * N * Fin + N * Fout + Fout)),
    )(x, weight, adj, b2)

    return out
